# Initial kernel scaffold; baseline (speedup 1.0000x reference)
import functools

import jax
import jax.numpy as jnp
from jax import lax
from jax.experimental import pallas as pl
from jax.experimental.pallas import tpu as pltpu
from jax.experimental.pallas import tpu_sc as plsc

B = 1024
K = 16
D = 32
NUM_REL = 32

_NC = 2
_NS = 16
_NW = _NC * _NS
_BPW = B // _NW

_sc_mesh = plsc.VectorSubcoreMesh(core_axis_name="c", subcore_axis_name="s")


@functools.partial(
    pl.kernel,
    out_type=[
        jax.ShapeDtypeStruct((B, D), jnp.float32),
        jax.ShapeDtypeStruct((B, D), jnp.float32),
        jax.ShapeDtypeStruct((B, K), jnp.int32),
        jax.ShapeDtypeStruct((B * K, D), jnp.float32),
    ],
    mesh=_sc_mesh,
    scratch_types=[
        pltpu.VMEM((_BPW,), jnp.int32),
        pltpu.VMEM((_BPW,), jnp.int32),
        pltpu.VMEM((_BPW, D), jnp.float32),
        pltpu.VMEM((_BPW, D), jnp.float32),
        pltpu.VMEM((_BPW, K), jnp.int32),
        pltpu.VMEM((_BPW, K), jnp.int32),
        pltpu.VMEM((_BPW * K, D), jnp.float32),
        pltpu.SemaphoreType.DMA,
    ],
)
def _sc_gather(u_hbm, v_hbm, adj_ent_hbm, adj_rel_hbm, usr_hbm, ent_hbm,
               ue_out, vs_out, rel_out, ne_out,
               u_v, v_v, ue_v, vs_v, nbr_v, rel_v, ne_v, sem):
    wid = lax.axis_index("s") * _NC + lax.axis_index("c")
    base = wid * _BPW
    pltpu.sync_copy(u_hbm.at[pl.ds(base, _BPW)], u_v)
    pltpu.sync_copy(v_hbm.at[pl.ds(base, _BPW)], v_v)

    c_ue = pltpu.async_copy(usr_hbm.at[u_v], ue_v, sem)
    c_vs = pltpu.async_copy(ent_hbm.at[v_v], vs_v, sem)
    c_nb = pltpu.async_copy(adj_ent_hbm.at[v_v], nbr_v, sem)
    c_rl = pltpu.async_copy(adj_rel_hbm.at[v_v], rel_v, sem)
    c_nb.wait()

    copies = []
    for i in range(_BPW):
        copies.append(
            pltpu.async_copy(ent_hbm.at[nbr_v.at[i]],
                             ne_v.at[pl.ds(i * K, K)], sem))
    c_ue.wait()
    c_vs.wait()
    c_rl.wait()
    for c in copies:
        c.wait()

    pltpu.sync_copy(ue_v, ue_out.at[pl.ds(base, _BPW)])
    pltpu.sync_copy(vs_v, vs_out.at[pl.ds(base, _BPW)])
    pltpu.sync_copy(rel_v, rel_out.at[pl.ds(base, _BPW)])
    pltpu.sync_copy(ne_v, ne_out.at[pl.ds(base * K, _BPW * K)])


def _tc_body(ue_ref, vs_ref, rel_ref, ne_ref, rel_emb_ref, w_ref, b_ref,
             out_ref):
    u_e = ue_ref[...]
    logits = lax.dot_general(u_e, rel_emb_ref[...],
                             (((1,), (1,)), ((), ())),
                             preferred_element_type=jnp.float32)
    rel = rel_ref[...]
    r_iota = lax.broadcasted_iota(jnp.int32, (B, K, NUM_REL), 2)
    onehot = rel[:, :, None] == r_iota
    scores = jnp.sum(jnp.where(onehot, logits[:, None, :], 0.0), axis=2)
    scores = jax.nn.softmax(scores, axis=1)
    n_e = ne_ref[...]
    e_u = jnp.sum(scores[:, :, None] * n_e, axis=1)
    h = lax.dot_general(e_u + vs_ref[...], w_ref[...],
                        (((1,), (1,)), ((), ())),
                        preferred_element_type=jnp.float32)
    v_u = jnp.maximum(h + b_ref[...], 0.0)
    out_ref[...] = jax.nn.sigmoid(
        jnp.sum(u_e * v_u, axis=1, keepdims=True))


_tc_call = pl.pallas_call(
    _tc_body,
    out_shape=jax.ShapeDtypeStruct((B, 1), jnp.float32),
)


@jax.jit
def kernel(u, v, adj_ent, adj_rel, usr_emb, ent_emb, rel_emb, W, b):
    u_e, v_self, rel, n_e = _sc_gather(u, v, adj_ent, adj_rel, usr_emb,
                                       ent_emb)
    out = _tc_call(u_e, v_self, rel, n_e.reshape(B, K, D), rel_emb, W,
                   b.reshape(1, D))
    return out.reshape(B)


# speedup vs baseline: 1.0591x; 1.0591x over previous
import functools

import jax
import jax.numpy as jnp
from jax import lax
from jax.experimental import pallas as pl
from jax.experimental.pallas import tpu as pltpu
from jax.experimental.pallas import tpu_sc as plsc

B = 1024
K = 16
D = 32
NUM_REL = 32

_NC = 2
_NS = 16
_NW = _NC * _NS
_BPW = B // _NW

def _sc_gather_body(u_hbm, v_hbm, adj_ent_hbm, adj_rel_hbm, usr_hbm, ent_hbm,
               ue_out, vs_out, rel_out, ne_out,
               u_v, v_v, ue_v, vs_v, nbr_v, nbr_flat, rel_v, ne_v, sem, sem2):
    wid = lax.axis_index("s") * _NC + lax.axis_index("c")
    base = wid * _BPW
    pltpu.sync_copy(u_hbm.at[pl.ds(base, _BPW)], u_v)
    pltpu.sync_copy(v_hbm.at[pl.ds(base, _BPW)], v_v)

    c_ue = pltpu.async_copy(usr_hbm.at[u_v], ue_v, sem)
    c_vs = pltpu.async_copy(ent_hbm.at[v_v], vs_v, sem)
    c_rl = pltpu.async_copy(adj_rel_hbm.at[v_v], rel_v, sem)
    c_nb = pltpu.async_copy(adj_ent_hbm.at[v_v], nbr_v, sem2)
    c_nb.wait()

    for i in range(_BPW):
        nbr_flat[pl.ds(i * K, K)] = nbr_v[i, :]

    chunk = 128
    copies = []
    for j in range(0, _BPW * K, chunk):
        copies.append(
            pltpu.async_copy(ent_hbm.at[nbr_flat.at[pl.ds(j, chunk)]],
                             ne_v.at[pl.ds(j, chunk)], sem2))
    c_ue.wait()
    c_vs.wait()
    c_rl.wait()
    for c in copies:
        c.wait()

    pltpu.sync_copy(ue_v, ue_out.at[pl.ds(base, _BPW)])
    pltpu.sync_copy(vs_v, vs_out.at[pl.ds(base, _BPW)])
    pltpu.sync_copy(rel_v, rel_out.at[pl.ds(base, _BPW)])
    pltpu.sync_copy(ne_v, ne_out.at[pl.ds(base * K, _BPW * K)])


@functools.cache
def _sc_gather_call():
    mesh = plsc.VectorSubcoreMesh(core_axis_name="c", subcore_axis_name="s",
                                  num_cores=_NC, num_subcores=_NS)
    return pl.kernel(
        _sc_gather_body,
        out_type=[
            jax.ShapeDtypeStruct((B, D), jnp.float32),
            jax.ShapeDtypeStruct((B, D), jnp.float32),
            jax.ShapeDtypeStruct((B, K), jnp.int32),
            jax.ShapeDtypeStruct((B * K, D), jnp.float32),
        ],
        mesh=mesh,
        scratch_types=[
            pltpu.VMEM((_BPW,), jnp.int32),
            pltpu.VMEM((_BPW,), jnp.int32),
            pltpu.VMEM((_BPW, D), jnp.float32),
            pltpu.VMEM((_BPW, D), jnp.float32),
            pltpu.VMEM((_BPW, K), jnp.int32),
            pltpu.VMEM((_BPW * K,), jnp.int32),
            pltpu.VMEM((_BPW, K), jnp.int32),
            pltpu.VMEM((_BPW * K, D), jnp.float32),
            pltpu.SemaphoreType.DMA,
            pltpu.SemaphoreType.DMA,
        ],
        compiler_params=pltpu.CompilerParams(use_tc_tiling_on_sc=False),
    )


def _tc_body(ue_ref, vs_ref, rel_ref, ne_ref, rel_emb_ref, w_ref, b_ref,
             out_ref):
    u_e = ue_ref[...]
    logits = lax.dot_general(u_e, rel_emb_ref[...],
                             (((1,), (1,)), ((), ())),
                             preferred_element_type=jnp.float32)
    rel = rel_ref[...]
    r_iota = lax.broadcasted_iota(jnp.int32, (B, K, NUM_REL), 2)
    onehot = rel[:, :, None] == r_iota
    scores = jnp.sum(jnp.where(onehot, logits[:, None, :], 0.0), axis=2)
    scores = jax.nn.softmax(scores, axis=1)
    n_e = ne_ref[...]
    e_u = jnp.sum(scores[:, :, None] * n_e, axis=1)
    h = lax.dot_general(e_u + vs_ref[...], w_ref[...],
                        (((1,), (1,)), ((), ())),
                        preferred_element_type=jnp.float32)
    v_u = jnp.maximum(h + b_ref[...], 0.0)
    out_ref[...] = jax.nn.sigmoid(
        jnp.sum(u_e * v_u, axis=1, keepdims=True))


_tc_call = pl.pallas_call(
    _tc_body,
    out_shape=jax.ShapeDtypeStruct((B, 1), jnp.float32),
)


@jax.jit
def kernel(u, v, adj_ent, adj_rel, usr_emb, ent_emb, rel_emb, W, b):
    u_e, v_self, rel, n_e = _sc_gather_call()(u, v, adj_ent, adj_rel,
                                              usr_emb, ent_emb)
    out = _tc_call(u_e, v_self, rel, n_e.reshape(B, K, D), rel_emb, W,
                   b.reshape(1, D))
    return out.reshape(B)
